# baseline (speedup 1.0000x reference)
import jax
import jax.numpy as jnp
from jax.experimental import pallas as pl
from jax.experimental.pallas import tpu as pltpu

_C1_WP = 32
_C1_IN = 1280
_C1_ACC = 1024
_C1_POOL = 896
_C2_WP = 18
_C2_IN = 512
_C2_ACC = 256
_C2_POOL = 232
_C2_OUT = 128

_BT = 8
_NP = 8
_FC_T = 256


def _params():
    return pltpu.CompilerParams(
        dimension_semantics=("parallel",),
        vmem_limit_bytes=100 * 1024 * 1024,
    )


def _conv_stage_kernel(x_ref, w1x_ref, b1_ref, sel1_ref, w2x_ref, b2_ref,
                       sel2_ref, o_ref, col1_scr, col2_scr):
    for p in range(_NP):
        for j in range(25):
            s = (j // 5) * _C1_WP + (j % 5)
            col1_scr[j * 8:(j + 1) * 8, p * _C1_ACC:(p + 1) * _C1_ACC] = \
                x_ref[p * _BT:(p + 1) * _BT, s:s + _C1_ACC]
    acc = jnp.dot(w1x_ref[...], col1_scr[...],
                  preferred_element_type=jnp.float32)
    a = jnp.maximum(acc + b1_ref[...], 0.0)
    ms = []
    for p in range(_NP):
        ap = a[:, p * _C1_ACC:p * _C1_ACC + _C1_POOL + _C1_WP + 1]
        pr = jnp.maximum(ap[:, 0:_C1_POOL + _C1_WP],
                         ap[:, 1:1 + _C1_POOL + _C1_WP])
        ms.append(jnp.maximum(pr[:, 0:_C1_POOL],
                              pr[:, _C1_WP:_C1_WP + _C1_POOL]))
    m = jnp.concatenate(ms, axis=0).astype(jnp.bfloat16)
    h1 = jnp.dot(m, sel1_ref[...],
                 preferred_element_type=jnp.float32)
    for p in range(_NP):
        for bi in range(_BT):
            cb = bi * 400
            xi = h1[p * 128 + bi * 16:p * 128 + (bi + 1) * 16, :]
            for j in range(25):
                s = (j // 5) * _C2_WP + (j % 5)
                col2_scr[cb + j * 16:cb + (j + 1) * 16,
                         p * _C2_ACC:(p + 1) * _C2_ACC] = xi[:, s:s + _C2_ACC]
    acc2 = jnp.dot(w2x_ref[...], col2_scr[...],
                   preferred_element_type=jnp.float32)
    a2 = jnp.maximum(acc2 + b2_ref[...], 0.0)
    m2s = []
    for p in range(_NP):
        ap = a2[:, p * _C2_ACC:p * _C2_ACC + _C2_POOL + _C2_WP + 1]
        pr = jnp.maximum(ap[:, 0:_C2_POOL + _C2_WP],
                         ap[:, 1:1 + _C2_POOL + _C2_WP])
        m2s.append(jnp.maximum(pr[:, 0:_C2_POOL],
                               pr[:, _C2_WP:_C2_WP + _C2_POOL]))
    m2 = jnp.concatenate(m2s, axis=0)
    o_ref[...] = jnp.dot(m2, sel2_ref[...],
                         preferred_element_type=jnp.float32
                         ).astype(jnp.bfloat16)


def _fc_kernel(x_ref, w1_ref, b1_ref, w2_ref, b2_ref, o_ref):
    h = jnp.dot(x_ref[...], w1_ref[...], preferred_element_type=jnp.float32)
    h = jnp.maximum(h + b1_ref[...], 0.0)
    z = jnp.dot(h, w2_ref[...], preferred_element_type=jnp.float32) + b2_ref[...]
    z = z - jnp.max(z, axis=1, keepdims=True)
    o_ref[...] = z - jnp.log(jnp.sum(jnp.exp(z), axis=1, keepdims=True))


def kernel(x, conv1_w, conv1_b, conv2_w, conv2_b,
           fc1_w, fc1_b, fc2_w, fc2_b, pool1_sel, pool2_sel):
    n = x.shape[0]
    bt = _BT * _NP
    n_pad = ((n + bt - 1) // bt) * bt
    xp = jnp.pad(x, ((0, n_pad - n), (0, 0), (2, 2), (2, 2)))
    x1 = xp.reshape(n_pad, _C1_WP * _C1_WP)
    x1 = jnp.pad(x1, ((0, 0), (0, _C1_IN - _C1_WP * _C1_WP)))

    eye = jnp.eye(_BT, dtype=jnp.float32)
    w1x = jnp.einsum('cj,bB->bcjB', conv1_w, eye).reshape(_BT * 16, 25 * _BT)
    b1t = jnp.tile(conv1_b, (_BT, 1))
    w2x = jnp.einsum('bB,ck->bcBk', eye, conv2_w).reshape(_BT * 32, _BT * 400)
    b2t = jnp.tile(conv2_b, (_BT, 1))

    g = n_pad // bt
    h2 = pl.pallas_call(
        _conv_stage_kernel,
        out_shape=jax.ShapeDtypeStruct((n_pad * 32, _C2_OUT), jnp.bfloat16),
        grid=(g,),
        in_specs=[
            pl.BlockSpec((bt, _C1_IN), lambda i: (i, 0)),
            pl.BlockSpec((_BT * 16, 25 * _BT), lambda i: (0, 0)),
            pl.BlockSpec((_BT * 16, 1), lambda i: (0, 0)),
            pl.BlockSpec((_C1_POOL, _C2_IN), lambda i: (0, 0)),
            pl.BlockSpec((_BT * 32, _BT * 400), lambda i: (0, 0)),
            pl.BlockSpec((_BT * 32, 1), lambda i: (0, 0)),
            pl.BlockSpec((_C2_POOL, _C2_OUT), lambda i: (0, 0)),
        ],
        out_specs=pl.BlockSpec((bt * 32, _C2_OUT), lambda i: (i, 0)),
        scratch_shapes=[
            pltpu.VMEM((25 * _BT, _NP * _C1_ACC), jnp.float32),
            pltpu.VMEM((_BT * 400, _NP * _C2_ACC), jnp.float32),
        ],
        compiler_params=_params(),
    )(x1, w1x, b1t, pool1_sel.astype(jnp.bfloat16), w2x, b2t, pool2_sel)

    feats = h2.reshape(n_pad, 32 * _C2_OUT)
    nf = ((n_pad + _FC_T - 1) // _FC_T) * _FC_T
    if nf != n_pad:
        feats = jnp.pad(feats, ((0, nf - n_pad), (0, 0)))
    out = pl.pallas_call(
        _fc_kernel,
        out_shape=jax.ShapeDtypeStruct((nf, 10), jnp.float32),
        grid=(nf // _FC_T,),
        in_specs=[
            pl.BlockSpec((_FC_T, 32 * _C2_OUT), lambda i: (i, 0)),
            pl.BlockSpec((32 * _C2_OUT, 128), lambda i: (0, 0)),
            pl.BlockSpec((1, 128), lambda i: (0, 0)),
            pl.BlockSpec((128, 10), lambda i: (0, 0)),
            pl.BlockSpec((1, 10), lambda i: (0, 0)),
        ],
        out_specs=pl.BlockSpec((_FC_T, 10), lambda i: (i, 0)),
        compiler_params=_params(),
    )(feats, fc1_w.astype(jnp.bfloat16), fc1_b, fc2_w, fc2_b)
    return out[:n]


# speedup vs baseline: 1.0088x; 1.0088x over previous
import jax
import jax.numpy as jnp
from jax.experimental import pallas as pl
from jax.experimental.pallas import tpu as pltpu

_C1_WP = 32
_C1_IN = 1280
_C1_ACC = 1024
_C1_POOL = 896
_C2_WP = 18
_C2_IN = 512
_C2_ACC = 256
_C2_POOL = 232
_C2_OUT = 128

_BT = 8
_NP = 8
_FC_T = 256


def _params():
    return pltpu.CompilerParams(
        dimension_semantics=("parallel",),
        vmem_limit_bytes=100 * 1024 * 1024,
    )


def _conv_stage_kernel(x_ref, w1x_ref, b1_ref, sel1_ref, w2x_ref, b2_ref,
                       sel2_ref, o_ref, col1_scr, col2_scr):
    for p in range(_NP):
        for j in range(25):
            s = (j // 5) * _C1_WP + (j % 5)
            col1_scr[j * 8:(j + 1) * 8, p * _C1_ACC:(p + 1) * _C1_ACC] = \
                x_ref[p * _BT:(p + 1) * _BT, s:s + _C1_ACC]
    acc = jnp.dot(w1x_ref[...], col1_scr[...],
                  preferred_element_type=jnp.float32)
    a = jnp.maximum(acc + b1_ref[...], 0.0)
    ms = []
    for p in range(_NP):
        ap = a[:, p * _C1_ACC:p * _C1_ACC + _C1_POOL + _C1_WP + 1]
        pr = jnp.maximum(ap[:, 0:_C1_POOL + _C1_WP],
                         ap[:, 1:1 + _C1_POOL + _C1_WP])
        ms.append(jnp.maximum(pr[:, 0:_C1_POOL],
                              pr[:, _C1_WP:_C1_WP + _C1_POOL]))
    m = jnp.concatenate(ms, axis=0)
    h1 = jnp.dot(m, sel1_ref[...],
                 preferred_element_type=jnp.float32)
    for p in range(_NP):
        for bi in range(_BT):
            cb = bi * 400
            xi = h1[p * 128 + bi * 16:p * 128 + (bi + 1) * 16, :]
            for j in range(25):
                s = (j // 5) * _C2_WP + (j % 5)
                col2_scr[cb + j * 16:cb + (j + 1) * 16,
                         p * _C2_ACC:(p + 1) * _C2_ACC] = xi[:, s:s + _C2_ACC]
    acc2 = jnp.dot(w2x_ref[...], col2_scr[...],
                   preferred_element_type=jnp.float32)
    a2 = jnp.maximum(acc2 + b2_ref[...], 0.0)
    m2s = []
    for p in range(_NP):
        ap = a2[:, p * _C2_ACC:p * _C2_ACC + _C2_POOL + _C2_WP + 1]
        pr = jnp.maximum(ap[:, 0:_C2_POOL + _C2_WP],
                         ap[:, 1:1 + _C2_POOL + _C2_WP])
        m2s.append(jnp.maximum(pr[:, 0:_C2_POOL],
                               pr[:, _C2_WP:_C2_WP + _C2_POOL]))
    m2 = jnp.concatenate(m2s, axis=0)
    o_ref[...] = jnp.dot(m2, sel2_ref[...],
                         preferred_element_type=jnp.float32
                         ).astype(jnp.bfloat16)


def _fc_kernel(x_ref, w1_ref, b1_ref, w2_ref, b2_ref, o_ref):
    h = jnp.dot(x_ref[...], w1_ref[...], preferred_element_type=jnp.float32)
    h = jnp.maximum(h + b1_ref[...], 0.0)
    z = jnp.dot(h, w2_ref[...], preferred_element_type=jnp.float32) + b2_ref[...]
    z = z - jnp.max(z, axis=1, keepdims=True)
    o_ref[...] = z - jnp.log(jnp.sum(jnp.exp(z), axis=1, keepdims=True))


def kernel(x, conv1_w, conv1_b, conv2_w, conv2_b,
           fc1_w, fc1_b, fc2_w, fc2_b, pool1_sel, pool2_sel):
    n = x.shape[0]
    bt = _BT * _NP
    n_pad = ((n + bt - 1) // bt) * bt
    xp = jnp.pad(x, ((0, n_pad - n), (0, 0), (2, 2), (2, 2)))
    x1 = xp.reshape(n_pad, _C1_WP * _C1_WP)
    x1 = jnp.pad(x1, ((0, 0), (0, _C1_IN - _C1_WP * _C1_WP)))

    eye = jnp.eye(_BT, dtype=jnp.float32)
    w1x = jnp.einsum('cj,bB->bcjB', conv1_w, eye).reshape(_BT * 16, 25 * _BT)
    b1t = jnp.tile(conv1_b, (_BT, 1))
    w2x = jnp.einsum('bB,ck->bcBk', eye, conv2_w).reshape(_BT * 32, _BT * 400)
    b2t = jnp.tile(conv2_b, (_BT, 1))

    g = n_pad // bt
    h2 = pl.pallas_call(
        _conv_stage_kernel,
        out_shape=jax.ShapeDtypeStruct((n_pad * 32, _C2_OUT), jnp.bfloat16),
        grid=(g,),
        in_specs=[
            pl.BlockSpec((bt, _C1_IN), lambda i: (i, 0)),
            pl.BlockSpec((_BT * 16, 25 * _BT), lambda i: (0, 0)),
            pl.BlockSpec((_BT * 16, 1), lambda i: (0, 0)),
            pl.BlockSpec((_C1_POOL, _C2_IN), lambda i: (0, 0)),
            pl.BlockSpec((_BT * 32, _BT * 400), lambda i: (0, 0)),
            pl.BlockSpec((_BT * 32, 1), lambda i: (0, 0)),
            pl.BlockSpec((_C2_POOL, _C2_OUT), lambda i: (0, 0)),
        ],
        out_specs=pl.BlockSpec((bt * 32, _C2_OUT), lambda i: (i, 0)),
        scratch_shapes=[
            pltpu.VMEM((25 * _BT, _NP * _C1_ACC), jnp.float32),
            pltpu.VMEM((_BT * 400, _NP * _C2_ACC), jnp.float32),
        ],
        compiler_params=_params(),
    )(x1, w1x, b1t, pool1_sel, w2x, b2t, pool2_sel)

    feats = h2.reshape(n_pad, 32 * _C2_OUT)
    nf = ((n_pad + _FC_T - 1) // _FC_T) * _FC_T
    if nf != n_pad:
        feats = jnp.pad(feats, ((0, nf - n_pad), (0, 0)))
    out = pl.pallas_call(
        _fc_kernel,
        out_shape=jax.ShapeDtypeStruct((nf, 10), jnp.float32),
        grid=(nf // _FC_T,),
        in_specs=[
            pl.BlockSpec((_FC_T, 32 * _C2_OUT), lambda i: (i, 0)),
            pl.BlockSpec((32 * _C2_OUT, 128), lambda i: (0, 0)),
            pl.BlockSpec((1, 128), lambda i: (0, 0)),
            pl.BlockSpec((128, 10), lambda i: (0, 0)),
            pl.BlockSpec((1, 10), lambda i: (0, 0)),
        ],
        out_specs=pl.BlockSpec((_FC_T, 10), lambda i: (i, 0)),
        compiler_params=_params(),
    )(feats, fc1_w.astype(jnp.bfloat16), fc1_b, fc2_w, fc2_b)
    return out[:n]


# speedup vs baseline: 1.0253x; 1.0164x over previous
import jax
import jax.numpy as jnp
from jax.experimental import pallas as pl
from jax.experimental.pallas import tpu as pltpu

_C1_WP = 32
_C1_IN = 1280
_C1_ACC = 1024
_C1_POOL = 896
_C2_WP = 18
_C2_IN = 512
_C2_ACC = 256
_C2_POOL = 232
_C2_OUT = 128

_BT = 8
_NP = 8
_FC_T = 256


def _params():
    return pltpu.CompilerParams(
        dimension_semantics=("parallel",),
        vmem_limit_bytes=100 * 1024 * 1024,
    )


def _conv_stage_kernel(x_ref, w1x_ref, b1_ref, sel1_ref, w2x_ref, b2_ref,
                       sel2_ref, o_ref, col1_scr, col2_scr):
    for p in range(_NP):
        for j in range(25):
            s = (j // 5) * _C1_WP + (j % 5)
            col1_scr[j * 8:(j + 1) * 8, p * _C1_ACC:(p + 1) * _C1_ACC] = \
                x_ref[p * _BT:(p + 1) * _BT, s:s + _C1_ACC]
    acc = jnp.dot(w1x_ref[...], col1_scr[...],
                  preferred_element_type=jnp.float32)
    a = jnp.maximum(acc + b1_ref[...], 0.0)
    ms = []
    for p in range(_NP):
        ap = a[:, p * _C1_ACC:p * _C1_ACC + _C1_POOL + _C1_WP + 1]
        pr = jnp.maximum(ap[:, 0:_C1_POOL + _C1_WP],
                         ap[:, 1:1 + _C1_POOL + _C1_WP])
        ms.append(jnp.maximum(pr[:, 0:_C1_POOL],
                              pr[:, _C1_WP:_C1_WP + _C1_POOL]))
    m = jnp.concatenate(ms, axis=0)
    h1 = jnp.dot(m, sel1_ref[...],
                 preferred_element_type=jnp.float32)
    for p in range(_NP):
        for bi in range(_BT):
            cb = bi * 400
            xi = h1[p * 128 + bi * 16:p * 128 + (bi + 1) * 16, :]
            for j in range(25):
                s = (j // 5) * _C2_WP + (j % 5)
                col2_scr[cb + j * 16:cb + (j + 1) * 16,
                         p * _C2_ACC:(p + 1) * _C2_ACC] = xi[:, s:s + _C2_ACC]
    acc2 = jnp.dot(w2x_ref[...], col2_scr[...],
                   preferred_element_type=jnp.float32)
    a2 = jnp.maximum(acc2 + b2_ref[...], 0.0)
    m2s = []
    for p in range(_NP):
        ap = a2[:, p * _C2_ACC:p * _C2_ACC + _C2_POOL + _C2_WP + 1]
        pr = jnp.maximum(ap[:, 0:_C2_POOL + _C2_WP],
                         ap[:, 1:1 + _C2_POOL + _C2_WP])
        m2s.append(jnp.maximum(pr[:, 0:_C2_POOL],
                               pr[:, _C2_WP:_C2_WP + _C2_POOL]))
    m2 = jnp.concatenate(m2s, axis=0)
    o_ref[...] = jnp.dot(m2, sel2_ref[...],
                         preferred_element_type=jnp.float32
                         ).astype(jnp.bfloat16)


def _fc_kernel(x_ref, w1_ref, b1_ref, w2_ref, b2_ref, o_ref):
    h = jnp.dot(x_ref[...], w1_ref[...], preferred_element_type=jnp.float32)
    h = jnp.maximum(h + b1_ref[...], 0.0)
    z = jnp.dot(h, w2_ref[...], preferred_element_type=jnp.float32) + b2_ref[...]
    z = z - jnp.max(z, axis=1, keepdims=True)
    o_ref[...] = z - jnp.log(jnp.sum(jnp.exp(z), axis=1, keepdims=True))


def kernel(x, conv1_w, conv1_b, conv2_w, conv2_b,
           fc1_w, fc1_b, fc2_w, fc2_b, pool1_sel, pool2_sel):
    n = x.shape[0]
    bt = _BT * _NP
    n_pad = ((n + bt - 1) // bt) * bt
    xp = jnp.pad(x, ((0, n_pad - n), (0, 0), (2, 2), (2, 2)))
    x1 = xp.reshape(n_pad, _C1_WP * _C1_WP)
    x1 = jnp.pad(x1, ((0, 0), (0, _C1_IN - _C1_WP * _C1_WP)))

    eye = jnp.eye(_BT, dtype=jnp.float32)
    w1x = jnp.einsum('cj,bB->bcjB', conv1_w, eye).reshape(_BT * 16, 25 * _BT)
    b1t = jnp.tile(conv1_b, (_BT, 1))
    w2x = jnp.einsum('bB,ck->bcBk', eye, conv2_w).reshape(_BT * 32, _BT * 400)
    b2t = jnp.tile(conv2_b, (_BT, 1))

    g = n_pad // bt
    h2 = pl.pallas_call(
        _conv_stage_kernel,
        out_shape=jax.ShapeDtypeStruct((n_pad * 32, _C2_OUT), jnp.bfloat16),
        grid=(g,),
        in_specs=[
            pl.BlockSpec((bt, _C1_IN), lambda i: (i, 0)),
            pl.BlockSpec((_BT * 16, 25 * _BT), lambda i: (0, 0)),
            pl.BlockSpec((_BT * 16, 1), lambda i: (0, 0)),
            pl.BlockSpec((_C1_POOL, _C2_IN), lambda i: (0, 0)),
            pl.BlockSpec((_BT * 32, _BT * 400), lambda i: (0, 0)),
            pl.BlockSpec((_BT * 32, 1), lambda i: (0, 0)),
            pl.BlockSpec((_C2_POOL, _C2_OUT), lambda i: (0, 0)),
        ],
        out_specs=pl.BlockSpec((bt * 32, _C2_OUT), lambda i: (i, 0)),
        scratch_shapes=[
            pltpu.VMEM((25 * _BT, _NP * _C1_ACC), jnp.float32),
            pltpu.VMEM((_BT * 400, _NP * _C2_ACC), jnp.float32),
        ],
        compiler_params=pltpu.CompilerParams(
            dimension_semantics=("parallel",),
            vmem_limit_bytes=100 * 1024 * 1024,
            allow_input_fusion=[True, False, False, False, False, False,
                                False],
        ),
    )(x1, w1x, b1t, pool1_sel, w2x, b2t, pool2_sel)

    feats = h2.reshape(n_pad, 32 * _C2_OUT)
    nf = ((n_pad + _FC_T - 1) // _FC_T) * _FC_T
    if nf != n_pad:
        feats = jnp.pad(feats, ((0, nf - n_pad), (0, 0)))
    out = pl.pallas_call(
        _fc_kernel,
        out_shape=jax.ShapeDtypeStruct((nf, 10), jnp.float32),
        grid=(nf // _FC_T,),
        in_specs=[
            pl.BlockSpec((_FC_T, 32 * _C2_OUT), lambda i: (i, 0)),
            pl.BlockSpec((32 * _C2_OUT, 128), lambda i: (0, 0)),
            pl.BlockSpec((1, 128), lambda i: (0, 0)),
            pl.BlockSpec((128, 10), lambda i: (0, 0)),
            pl.BlockSpec((1, 10), lambda i: (0, 0)),
        ],
        out_specs=pl.BlockSpec((_FC_T, 10), lambda i: (i, 0)),
        compiler_params=_params(),
    )(feats, fc1_w.astype(jnp.bfloat16), fc1_b, fc2_w, fc2_b)
    return out[:n]


# speedup vs baseline: 1.0255x; 1.0001x over previous
import jax
import jax.numpy as jnp
from jax.experimental import pallas as pl
from jax.experimental.pallas import tpu as pltpu

_C1_WP = 32
_C1_IN = 1280
_C1_ACC = 1024
_C1_POOL = 896
_C2_WP = 18
_C2_IN = 512
_C2_ACC = 256
_C2_POOL = 232
_C2_OUT = 128

_BT = 8
_NP = 8
_FC_T = 256


def _params():
    return pltpu.CompilerParams(
        dimension_semantics=("parallel",),
        vmem_limit_bytes=100 * 1024 * 1024,
    )


def _conv_stage_kernel(x_ref, w1x_ref, b1_ref, sel1_ref, w2x_ref, b2_ref,
                       sel2_ref, o_ref, col1_scr, col2_scr):
    for p in range(_NP):
        for j in range(25):
            s = (j // 5) * _C1_WP + (j % 5)
            col1_scr[j * 8:(j + 1) * 8, p * _C1_ACC:(p + 1) * _C1_ACC] = \
                x_ref[p * _BT:(p + 1) * _BT, s:s + _C1_ACC]
    acc = jnp.dot(w1x_ref[...], col1_scr[...],
                  preferred_element_type=jnp.float32)
    a = jnp.maximum(acc + b1_ref[...], 0.0)
    ms = []
    for p in range(_NP):
        ap = a[:, p * _C1_ACC:p * _C1_ACC + _C1_POOL + _C1_WP + 1]
        pr = jnp.maximum(ap[:, 0:_C1_POOL + _C1_WP],
                         ap[:, 1:1 + _C1_POOL + _C1_WP])
        ms.append(jnp.maximum(pr[:, 0:_C1_POOL],
                              pr[:, _C1_WP:_C1_WP + _C1_POOL]))
    m = jnp.concatenate(ms, axis=0)
    h1 = jnp.dot(m, sel1_ref[...],
                 preferred_element_type=jnp.float32)
    for p in range(_NP):
        for bi in range(_BT):
            cb = bi * 400
            xi = h1[p * 128 + bi * 16:p * 128 + (bi + 1) * 16, :]
            for j in range(25):
                s = (j // 5) * _C2_WP + (j % 5)
                col2_scr[cb + j * 16:cb + (j + 1) * 16,
                         p * _C2_ACC:(p + 1) * _C2_ACC] = xi[:, s:s + _C2_ACC]
    acc2 = jnp.dot(w2x_ref[...], col2_scr[...],
                   preferred_element_type=jnp.float32)
    a2 = jnp.maximum(acc2 + b2_ref[...], 0.0)
    m2s = []
    for p in range(_NP):
        ap = a2[:, p * _C2_ACC:p * _C2_ACC + _C2_POOL + _C2_WP + 1]
        pr = jnp.maximum(ap[:, 0:_C2_POOL + _C2_WP],
                         ap[:, 1:1 + _C2_POOL + _C2_WP])
        m2s.append(jnp.maximum(pr[:, 0:_C2_POOL],
                               pr[:, _C2_WP:_C2_WP + _C2_POOL]))
    m2 = jnp.concatenate(m2s, axis=0)
    o_ref[...] = jnp.dot(m2, sel2_ref[...],
                         preferred_element_type=jnp.float32
                         ).astype(jnp.bfloat16)


def _fc_kernel(x_ref, w1_ref, b1_ref, w2_ref, b2_ref, o_ref):
    h = jnp.dot(x_ref[...], w1_ref[...], preferred_element_type=jnp.float32)
    h = jnp.maximum(h + b1_ref[...], 0.0)
    z = jnp.dot(h, w2_ref[...], preferred_element_type=jnp.float32) + b2_ref[...]
    z = z - jnp.max(z, axis=1, keepdims=True)
    o_ref[...] = z - jnp.log(jnp.sum(jnp.exp(z), axis=1, keepdims=True))


def kernel(x, conv1_w, conv1_b, conv2_w, conv2_b,
           fc1_w, fc1_b, fc2_w, fc2_b, pool1_sel, pool2_sel):
    n = x.shape[0]
    bt = _BT * _NP
    n_pad = ((n + bt - 1) // bt) * bt
    xp = jnp.pad(x, ((0, n_pad - n), (0, 0), (2, 2), (2, 2)))
    x1 = xp.reshape(n_pad, _C1_WP * _C1_WP)
    x1 = jnp.pad(x1, ((0, 0), (0, _C1_IN - _C1_WP * _C1_WP)))

    eye = jnp.eye(_BT, dtype=jnp.float32)
    w1x = jnp.einsum('cj,bB->bcjB', conv1_w, eye).reshape(_BT * 16, 25 * _BT)
    b1t = jnp.tile(conv1_b, (_BT, 1))
    w2x = jnp.einsum('bB,ck->bcBk', eye, conv2_w).reshape(_BT * 32, _BT * 400)
    b2t = jnp.tile(conv2_b, (_BT, 1))

    g = n_pad // bt
    h2 = pl.pallas_call(
        _conv_stage_kernel,
        out_shape=jax.ShapeDtypeStruct((n_pad * 32, _C2_OUT), jnp.bfloat16),
        grid=(g,),
        in_specs=[
            pl.BlockSpec((bt, _C1_IN), lambda i: (i, 0)),
            pl.BlockSpec((_BT * 16, 25 * _BT), lambda i: (0, 0)),
            pl.BlockSpec((_BT * 16, 1), lambda i: (0, 0)),
            pl.BlockSpec((_C1_POOL, _C2_IN), lambda i: (0, 0)),
            pl.BlockSpec((_BT * 32, _BT * 400), lambda i: (0, 0)),
            pl.BlockSpec((_BT * 32, 1), lambda i: (0, 0)),
            pl.BlockSpec((_C2_POOL, _C2_OUT), lambda i: (0, 0)),
        ],
        out_specs=pl.BlockSpec((bt * 32, _C2_OUT), lambda i: (i, 0)),
        scratch_shapes=[
            pltpu.VMEM((25 * _BT, _NP * _C1_ACC), jnp.float32),
            pltpu.VMEM((_BT * 400, _NP * _C2_ACC), jnp.float32),
        ],
        compiler_params=pltpu.CompilerParams(
            dimension_semantics=("parallel",),
            vmem_limit_bytes=100 * 1024 * 1024,
            allow_input_fusion=[True, False, False, False, False, False,
                                False],
        ),
    )(x1, w1x, b1t, pool1_sel, w2x, b2t, pool2_sel)

    feats = h2.reshape(n_pad, 32 * _C2_OUT)
    nf = ((n_pad + _FC_T - 1) // _FC_T) * _FC_T
    if nf != n_pad:
        feats = jnp.pad(feats, ((0, nf - n_pad), (0, 0)))
    out = pl.pallas_call(
        _fc_kernel,
        out_shape=jax.ShapeDtypeStruct((nf, 10), jnp.float32),
        grid=(nf // _FC_T,),
        in_specs=[
            pl.BlockSpec((_FC_T, 32 * _C2_OUT), lambda i: (i, 0)),
            pl.BlockSpec((32 * _C2_OUT, 128), lambda i: (0, 0)),
            pl.BlockSpec((1, 128), lambda i: (0, 0)),
            pl.BlockSpec((128, 10), lambda i: (0, 0)),
            pl.BlockSpec((1, 10), lambda i: (0, 0)),
        ],
        out_specs=pl.BlockSpec((_FC_T, 10), lambda i: (i, 0)),
        compiler_params=pltpu.CompilerParams(
            dimension_semantics=("parallel",),
            vmem_limit_bytes=100 * 1024 * 1024,
            allow_input_fusion=[True, False, False, False, False],
        ),
    )(feats, fc1_w.astype(jnp.bfloat16), fc1_b, fc2_w, fc2_b)
    return out[:n]


# speedup vs baseline: 1.0256x; 1.0002x over previous
import jax
import jax.numpy as jnp
from jax.experimental import pallas as pl
from jax.experimental.pallas import tpu as pltpu

_C1_WP = 32
_C1_IN = 1280
_C1_ACC = 1024
_C1_POOL = 896
_C2_WP = 18
_C2_IN = 512
_C2_ACC = 256
_C2_POOL = 232
_C2_OUT = 128

_BT = 8
_NP = 8
_FC_T = 256


def _params():
    return pltpu.CompilerParams(
        dimension_semantics=("parallel",),
        vmem_limit_bytes=100 * 1024 * 1024,
    )


def _conv_stage_kernel(x_ref, w1x_ref, b1_ref, sel1_ref, w2x_ref, b2_ref,
                       sel2_ref, o_ref, col1_scr, col2_scr):
    for p in range(_NP):
        for j in range(25):
            s = (j // 5) * _C1_WP + (j % 5)
            col1_scr[j * 8:(j + 1) * 8, p * _C1_ACC:(p + 1) * _C1_ACC] = \
                x_ref[p * _BT:(p + 1) * _BT, s:s + _C1_ACC]
    acc = jnp.dot(w1x_ref[...], col1_scr[...],
                  preferred_element_type=jnp.float32)
    a = jnp.maximum(acc + b1_ref[...], 0.0)
    ms = []
    for p in range(_NP):
        ap = a[:, p * _C1_ACC:p * _C1_ACC + _C1_POOL + _C1_WP + 1]
        pr = jnp.maximum(ap[:, 0:_C1_POOL + _C1_WP],
                         ap[:, 1:1 + _C1_POOL + _C1_WP])
        ms.append(jnp.maximum(pr[:, 0:_C1_POOL],
                              pr[:, _C1_WP:_C1_WP + _C1_POOL]))
    m = jnp.concatenate(ms, axis=0)
    h1 = jnp.dot(m, sel1_ref[...],
                 preferred_element_type=jnp.float32)
    for p in range(_NP):
        for bi in range(_BT):
            cb = bi * 400
            xi = h1[p * 128 + bi * 16:p * 128 + (bi + 1) * 16, :]
            for j in range(25):
                s = (j // 5) * _C2_WP + (j % 5)
                col2_scr[cb + j * 16:cb + (j + 1) * 16,
                         p * _C2_ACC:(p + 1) * _C2_ACC] = xi[:, s:s + _C2_ACC]
    acc2 = jnp.dot(w2x_ref[...], col2_scr[...],
                   preferred_element_type=jnp.float32)
    a2 = jnp.maximum(acc2 + b2_ref[...], 0.0)
    m2s = []
    for p in range(_NP):
        ap = a2[:, p * _C2_ACC:p * _C2_ACC + _C2_POOL + _C2_WP + 1]
        pr = jnp.maximum(ap[:, 0:_C2_POOL + _C2_WP],
                         ap[:, 1:1 + _C2_POOL + _C2_WP])
        m2s.append(jnp.maximum(pr[:, 0:_C2_POOL],
                               pr[:, _C2_WP:_C2_WP + _C2_POOL]))
    m2 = jnp.concatenate(m2s, axis=0)
    o_ref[...] = jnp.dot(m2, sel2_ref[...],
                         preferred_element_type=jnp.float32
                         ).astype(jnp.bfloat16)


def _fc_kernel(x_ref, w1_ref, b1_ref, w2_ref, b2_ref, o_ref):
    h = jnp.dot(x_ref[...], w1_ref[...], preferred_element_type=jnp.float32)
    h = jnp.maximum(h + b1_ref[...], 0.0)
    z = jnp.dot(h, w2_ref[...], preferred_element_type=jnp.float32) + b2_ref[...]
    z = z - jnp.max(z, axis=1, keepdims=True)
    o_ref[...] = z - jnp.log(jnp.sum(jnp.exp(z), axis=1, keepdims=True))


def kernel(x, conv1_w, conv1_b, conv2_w, conv2_b,
           fc1_w, fc1_b, fc2_w, fc2_b, pool1_sel, pool2_sel):
    n = x.shape[0]
    bt = _BT * _NP
    n_pad = ((n + bt - 1) // bt) * bt
    xp = jnp.pad(x, ((0, n_pad - n), (0, 0), (2, 2), (2, 2)))
    x1 = xp.reshape(n_pad, _C1_WP * _C1_WP)
    x1 = jnp.pad(x1, ((0, 0), (0, _C1_IN - _C1_WP * _C1_WP)))

    eye = jnp.eye(_BT, dtype=jnp.float32)
    w1x = jnp.einsum('cj,bB->bcjB', conv1_w, eye).reshape(_BT * 16, 25 * _BT)
    b1t = jnp.tile(conv1_b, (_BT, 1))
    w2x = jnp.einsum('bB,ck->bcBk', eye, conv2_w).reshape(_BT * 32, _BT * 400)
    b2t = jnp.tile(conv2_b, (_BT, 1))

    g = n_pad // bt
    h2 = pl.pallas_call(
        _conv_stage_kernel,
        out_shape=jax.ShapeDtypeStruct((n_pad * 32, _C2_OUT), jnp.bfloat16),
        grid=(g,),
        in_specs=[
            pl.BlockSpec((bt, _C1_IN), lambda i: (i, 0)),
            pl.BlockSpec((_BT * 16, 25 * _BT), lambda i: (0, 0)),
            pl.BlockSpec((_BT * 16, 1), lambda i: (0, 0)),
            pl.BlockSpec((_C1_POOL, _C2_IN), lambda i: (0, 0)),
            pl.BlockSpec((_BT * 32, _BT * 400), lambda i: (0, 0)),
            pl.BlockSpec((_BT * 32, 1), lambda i: (0, 0)),
            pl.BlockSpec((_C2_POOL, _C2_OUT), lambda i: (0, 0)),
        ],
        out_specs=pl.BlockSpec((bt * 32, _C2_OUT), lambda i: (i, 0)),
        scratch_shapes=[
            pltpu.VMEM((25 * _BT, _NP * _C1_ACC), jnp.float32),
            pltpu.VMEM((_BT * 400, _NP * _C2_ACC), jnp.float32),
        ],
        compiler_params=pltpu.CompilerParams(
            dimension_semantics=("parallel",),
            vmem_limit_bytes=100 * 1024 * 1024,
            allow_input_fusion=[True, False, False, False, False, False,
                                False],
        ),
    )(x1, w1x, b1t, pool1_sel, w2x, b2t, pool2_sel)

    feats = h2.reshape(n_pad, 32 * _C2_OUT)
    nf = ((n_pad + _FC_T - 1) // _FC_T) * _FC_T
    if nf != n_pad:
        feats = jnp.pad(feats, ((0, nf - n_pad), (0, 0)))
    out = pl.pallas_call(
        _fc_kernel,
        out_shape=jax.ShapeDtypeStruct((nf, 10), jnp.float32),
        grid=(nf // _FC_T,),
        in_specs=[
            pl.BlockSpec((_FC_T, 32 * _C2_OUT), lambda i: (i, 0)),
            pl.BlockSpec((32 * _C2_OUT, 128), lambda i: (0, 0)),
            pl.BlockSpec((1, 128), lambda i: (0, 0)),
            pl.BlockSpec((128, 10), lambda i: (0, 0)),
            pl.BlockSpec((1, 10), lambda i: (0, 0)),
        ],
        out_specs=pl.BlockSpec((_FC_T, 10), lambda i: (i, 0)),
        compiler_params=_params(),
    )(feats, fc1_w.astype(jnp.bfloat16), fc1_b, fc2_w, fc2_b)
    return out[:n]
